# baseline (speedup 1.0000x reference)
import functools

import jax
import jax.numpy as jnp
from jax.experimental import pallas as pl

_NN = 10
_SIGMA = 5.0
_BIG = 3.0e38
_PAD_D2 = 1.0e36
_LANES = 128
_NCHUNK = 3


def _nnencode_block(pts_ref, ccb_ref, out_ref, *, strip_rows):
    block_rows = pts_ref.shape[1]
    k_out = out_ref.shape[2]
    pts = pts_ref[0]
    dfull = jnp.dot(pts, ccb_ref[...],
                    preferred_element_type=jnp.float32)

    scale = -1.0 / (2.0 * _SIGMA ** 2)
    for si in range(block_rows // strip_rows):
        r0 = si * strip_rows
        d = jnp.maximum(dfull[r0:r0 + strip_rows], 0.0)
        chunks = [d[:, j * _LANES:(j + 1) * _LANES] for j in range(_NCHUNK)]
        d0, d1, d2c = chunks

        lo = jnp.minimum(d0, d1)
        hi = jnp.maximum(d0, d1)
        a = jnp.minimum(lo, d2c)
        b = jnp.maximum(lo, jnp.minimum(hi, d2c))
        c = jnp.maximum(hi, d2c)

        thr = None
        for _ in range(_NN):
            thr = jnp.min(a, axis=1, keepdims=True)
            pop = a <= thr
            a = jnp.where(pop, b, a)
            b = jnp.where(pop, c, b)
            c = jnp.where(pop, _BIG, c)

        keeps = [cj <= thr for cj in chunks]
        ws = [jnp.where(keeps[j], jnp.exp(chunks[j] * scale), 0.0)
              for j in range(_NCHUNK)]
        s = ws[0].sum(axis=1, keepdims=True)
        for j in range(1, _NCHUNK):
            s = s + ws[j].sum(axis=1, keepdims=True)
        inv = 1.0 / s
        full = jnp.concatenate(
            [jnp.where(keeps[j], ws[j] * inv, 0.0) for j in range(_NCHUNK)],
            axis=1)
        out_ref[0, r0:r0 + strip_rows, :] = full[:, :k_out]


@functools.partial(jax.jit,
                   static_argnames=("block_rows", "strip_rows", "interpret"))
def _nnencode(pts_nd, cc, block_rows=512, strip_rows=64, interpret=False):
    B, S, D = pts_nd.shape
    K = cc.shape[0]
    kp = _NCHUNK * _LANES
    x = pts_nd[..., 0]
    y = pts_nd[..., 1]
    pts_aug = jnp.stack(
        [x, y, x * x + y * y, jnp.ones_like(x)], axis=-1)
    cx = cc[:, 0]
    cy = cc[:, 1]
    ccb = jnp.zeros((4, kp), jnp.float32)
    ccb = ccb.at[0, :K].set(-2.0 * cx)
    ccb = ccb.at[1, :K].set(-2.0 * cy)
    ccb = ccb.at[2, :].set(1.0)
    ccb = ccb.at[3, :K].set(cx * cx + cy * cy)
    ccb = ccb.at[3, K:].set(_PAD_D2)
    grid = (B, S // block_rows)
    body = functools.partial(_nnencode_block, strip_rows=strip_rows)
    return pl.pallas_call(
        body,
        grid=grid,
        in_specs=[
            pl.BlockSpec((1, block_rows, 4), lambda i, j: (i, j, 0)),
            pl.BlockSpec((4, kp), lambda i, j: (0, 0)),
        ],
        out_specs=pl.BlockSpec((1, block_rows, K), lambda i, j: (i, j, 0)),
        out_shape=jax.ShapeDtypeStruct((B, S, K), jnp.float32),
        interpret=interpret,
    )(pts_aug, ccb)


def kernel(pts_nd, cc):
    return _nnencode(pts_nd, cc)
